# Initial kernel scaffold; baseline (speedup 1.0000x reference)
import jax
import jax.numpy as jnp
from jax.experimental import pallas as pl


def kernel(inputs):
    raise NotImplementedError("write your pallas kernel here")


# speedup vs baseline: 1.8228x; 1.8228x over previous
import jax
import jax.numpy as jnp
from jax.experimental import pallas as pl
from jax.experimental.pallas import tpu as pltpu

N_FIELDS = 26
N_PAIRS = N_FIELDS * (N_FIELDS - 1) // 2
BB = 64


def _pair_body(in_ref, out_ref):
    x = in_ref[...]
    off = 0
    for i in range(N_FIELDS - 1):
        w = N_FIELDS - 1 - i
        out_ref[:, off:off + w, :] = x[:, i:i + 1, :] * x[:, i + 1:, :]
        off += w


def kernel(inputs):
    b, f, d = inputs.shape
    grid = (b // BB,)
    return pl.pallas_call(
        _pair_body,
        grid=grid,
        in_specs=[pl.BlockSpec((BB, f, d), lambda g: (g, 0, 0))],
        out_specs=pl.BlockSpec((BB, N_PAIRS, d), lambda g: (g, 0, 0)),
        out_shape=jax.ShapeDtypeStruct((b, N_PAIRS, d), jnp.float32),
    )(inputs)


# speedup vs baseline: 1.8295x; 1.0037x over previous
import jax
import jax.numpy as jnp
from jax.experimental import pallas as pl
from jax.experimental.pallas import tpu as pltpu

N_FIELDS = 26
N_PAIRS = N_FIELDS * (N_FIELDS - 1) // 2
BB = 128


def _pair_body(in_ref, out_ref):
    x = in_ref[...]
    off = 0
    for i in range(N_FIELDS - 1):
        w = N_FIELDS - 1 - i
        out_ref[:, off:off + w, :] = x[:, i:i + 1, :] * x[:, i + 1:, :]
        off += w


def kernel(inputs):
    b, f, d = inputs.shape
    grid = (b // BB,)
    return pl.pallas_call(
        _pair_body,
        grid=grid,
        in_specs=[pl.BlockSpec((BB, f, d), lambda g: (g, 0, 0))],
        out_specs=pl.BlockSpec((BB, N_PAIRS, d), lambda g: (g, 0, 0)),
        out_shape=jax.ShapeDtypeStruct((b, N_PAIRS, d), jnp.float32),
    )(inputs)
